# Initial kernel scaffold; baseline (speedup 1.0000x reference)
import jax
import jax.numpy as jnp
from jax.experimental import pallas as pl


def kernel(x, pos, vel, edge_index, We1, be1, We2, be2, We3, be3, Wv1, bv1, Wv2, bv2, Wh1, bh1, Wh2, bh2):
    raise NotImplementedError("write your pallas kernel here")


# speedup vs baseline: 4.7858x; 4.7858x over previous
import functools

import jax
import jax.numpy as jnp
from jax import lax
from jax.experimental import pallas as pl
from jax.experimental.pallas import tpu as pltpu
from jax.experimental.pallas import tpu_sc as plsc

N = 10000
E = 320000
D = 128
H = 32

NC = 2
NS = 16
NW = NC * NS

TW = 80
SW = 40

EPW = E // NW
CH = 80
SUB = 5
BLK = CH * SUB
NBLK = EPW // BLK

BE = 8000


def _softplus(z):
    return jnp.maximum(z, 0.0) + jnp.log1p(jnp.exp(-jnp.abs(z)))


def _proj_body(x_ref, pos_ref, vel_ref, ws_ref, wd_ref, ts_ref, td_ref):
    xx = x_ref[...]
    geom = jnp.concatenate(
        [pos_ref[...], vel_ref[...], jnp.zeros((xx.shape[0], TW - 68), jnp.float32)],
        axis=1)
    ts_ref[...] = jnp.concatenate(
        [jnp.dot(xx, ws_ref[...], preferred_element_type=jnp.float32), geom], axis=1)
    td_ref[...] = jnp.concatenate(
        [jnp.dot(xx, wd_ref[...], preferred_element_type=jnp.float32), geom], axis=1)


def _project_nodes(x, pos, vel, ws, wd):
    return pl.pallas_call(
        _proj_body,
        out_shape=(jax.ShapeDtypeStruct((N, TW), jnp.float32),
                   jax.ShapeDtypeStruct((N, TW), jnp.float32)),
    )(x, pos, vel, ws, wd)


def _sc_gather_body(ts_hbm, td_hbm, src_hbm, dst_hbm, gs_hbm, gd_hbm,
                    srcv, dstv, bs, bd, sem_s, sem_d):
    wid = lax.axis_index("s") * NC + lax.axis_index("c")
    base = wid * EPW
    pltpu.sync_copy(src_hbm.at[pl.ds(base, EPW)], srcv)
    pltpu.sync_copy(dst_hbm.at[pl.ds(base, EPW)], dstv)

    @pl.loop(0, NBLK)
    def _blk(b):
        off = b * BLK
        cps = []
        for i in range(SUB):
            cps.append(pltpu.async_copy(
                ts_hbm.at[srcv.at[pl.ds(off + i * CH, CH)]],
                bs.at[pl.ds(i * CH, CH)], sem_s))
            cps.append(pltpu.async_copy(
                td_hbm.at[dstv.at[pl.ds(off + i * CH, CH)]],
                bd.at[pl.ds(i * CH, CH)], sem_d))
        for c in cps:
            c.wait()
        pltpu.sync_copy(bs, gs_hbm.at[pl.ds(base + off, BLK)])
        pltpu.sync_copy(bd, gd_hbm.at[pl.ds(base + off, BLK)])


_sc_gather = functools.partial(
    pl.kernel,
    out_type=(jax.ShapeDtypeStruct((E, TW), jnp.float32),
              jax.ShapeDtypeStruct((E, TW), jnp.float32)),
    mesh=plsc.VectorSubcoreMesh(core_axis_name="c", subcore_axis_name="s"),
    scratch_types=[
        pltpu.VMEM((EPW,), jnp.int32),
        pltpu.VMEM((EPW,), jnp.int32),
        pltpu.VMEM((BLK, TW), jnp.float32),
        pltpu.VMEM((BLK, TW), jnp.float32),
        pltpu.SemaphoreType.DMA,
        pltpu.SemaphoreType.DMA,
    ],
    compiler_params=pltpu.CompilerParams(use_tc_tiling_on_sc=False),
)(_sc_gather_body)


def _edge_body(gs_ref, gd_ref, w1d_ref, w1v_ref, be1_ref, we2_ref, be2_ref,
               we3_ref, be3_ref, wv1d_ref, wv1v_ref, bv1_ref, wv2_ref,
               bv2_ref, out_ref):
    gs = gs_ref[...]
    gd = gd_ref[...]
    rx = gs[:, 64:65] - gd[:, 64:65]
    ry = gs[:, 65:66] - gd[:, 65:66]
    wx = gs[:, 66:67] - gd[:, 66:67]
    wy = gs[:, 67:68] - gd[:, 67:68]
    dist = rx * rx + ry * ry
    dot = wx * rx + wy * ry
    e1 = (gd[:, 0:32] + gs[:, 0:32] + dist * w1d_ref[...] + dot * w1v_ref[...]
          + be1_ref[...])
    h1 = _softplus(e1)
    h2 = _softplus(jnp.dot(h1, we2_ref[...], preferred_element_type=jnp.float32)
                   + be2_ref[...])
    m_h = jnp.dot(h2, we3_ref[...], preferred_element_type=jnp.float32) + be3_ref[...]
    v1 = _softplus(gd[:, 32:64] + gs[:, 32:64] + dist * wv1d_ref[...]
                   + dot * wv1v_ref[...] + bv1_ref[...])
    vw = jnp.dot(v1, wv2_ref[...], preferred_element_type=jnp.float32) + bv2_ref[...]
    ones = jnp.ones_like(vw)
    pad = jnp.zeros((gs.shape[0], SW - 35), jnp.float32)
    out_ref[...] = jnp.concatenate([m_h, vw * rx, vw * ry, ones, pad], axis=1)


def _edge_mlp(gs, gd, w1d, w1v, be1, we2, be2, we3, be3, wv1d, wv1v, bv1,
              wv2, bv2):
    full = lambda s: pl.BlockSpec(s, lambda i: (0,) * len(s))
    return pl.pallas_call(
        _edge_body,
        grid=(E // BE,),
        in_specs=[
            pl.BlockSpec((BE, TW), lambda i: (i, 0)),
            pl.BlockSpec((BE, TW), lambda i: (i, 0)),
            full((1, H)), full((1, H)), full((1, H)),
            full((H, H)), full((1, H)),
            full((H, H)), full((1, H)),
            full((1, H)), full((1, H)), full((1, H)),
            full((H, 1)), full((1, 1)),
        ],
        out_specs=pl.BlockSpec((BE, SW), lambda i: (i, 0)),
        out_shape=jax.ShapeDtypeStruct((E, SW), jnp.float32),
    )(gs, gd, w1d, w1v, be1, we2, be2, we3, be3, wv1d, wv1v, bv1, wv2, bv2)


NROWS_PS = N // NS


def _sc_scatter_body(s_hbm, dsti_hbm, zeros_hbm, agg_hbm, idxv, sv, shared, sem):
    cid = lax.axis_index("c")
    sid = lax.axis_index("s")
    wid = sid * NC + cid
    pltpu.sync_copy(zeros_hbm.at[pl.ds(sid * NROWS_PS, NROWS_PS)],
                    shared.at[pl.ds(sid * NROWS_PS, NROWS_PS)])
    plsc.subcore_barrier()
    base = wid * EPW
    pltpu.sync_copy(dsti_hbm.at[wid], idxv)

    @pl.loop(0, NBLK)
    def _blk(b):
        pltpu.sync_copy(s_hbm.at[pl.ds(base + b * BLK, BLK)], sv)
        for i in range(SUB):
            pltpu.sync_copy(sv.at[pl.ds(i * CH, CH)],
                            shared.at[idxv.at[b * SUB + i]], add=True)

    plsc.subcore_barrier()
    pltpu.sync_copy(shared.at[pl.ds(sid * NROWS_PS, NROWS_PS)],
                    agg_hbm.at[cid, pl.ds(sid * NROWS_PS, NROWS_PS)])


_sc_scatter = functools.partial(
    pl.kernel,
    out_type=jax.ShapeDtypeStruct((NC, N, SW), jnp.float32),
    mesh=plsc.VectorSubcoreMesh(core_axis_name="c", subcore_axis_name="s"),
    scratch_types=[
        pltpu.VMEM((EPW // CH, CH), jnp.int32),
        pltpu.VMEM((BLK, SW), jnp.float32),
        pltpu.VMEM_SHARED((N, SW), jnp.float32),
        pltpu.SemaphoreType.DMA,
    ],
    compiler_params=pltpu.CompilerParams(use_tc_tiling_on_sc=False),
)(_sc_scatter_body)


def _node_body(x_ref, agg_ref, wh1x_ref, wh1m_ref, wh1n_ref, bh1_ref,
               wh2_ref, bh2_ref, out_ref):
    agg = agg_ref[0] + agg_ref[1]
    cnt = jnp.maximum(agg[:, 34:35], 1.0)
    mh = agg[:, 0:32] / cnt
    mvx = agg[:, 32:33] / cnt
    mvy = agg[:, 33:34] / cnt
    norm = jnp.sqrt(mvx * mvx + mvy * mvy + 1e-12)
    xx = x_ref[...]
    pre = (jnp.dot(xx, wh1x_ref[...], preferred_element_type=jnp.float32)
           + jnp.dot(mh, wh1m_ref[...], preferred_element_type=jnp.float32)
           + norm * wh1n_ref[...] + bh1_ref[...])
    hu = _softplus(pre)
    out_ref[...] = (xx + jnp.dot(hu, wh2_ref[...], preferred_element_type=jnp.float32)
                    + bh2_ref[...])


def _node_update(x, agg, wh1x, wh1m, wh1n, bh1, wh2, bh2):
    return pl.pallas_call(
        _node_body,
        out_shape=jax.ShapeDtypeStruct((N, D), jnp.float32),
    )(x, agg, wh1x, wh1m, wh1n, bh1, wh2, bh2)


def kernel(x, pos, vel, edge_index, We1, be1, We2, be2, We3, be3,
           Wv1, bv1, Wv2, bv2, Wh1, bh1, Wh2, bh2):
    src = edge_index[0]
    dst = edge_index[1]

    wd = jnp.concatenate([We1[:D], Wv1[:D]], axis=1)
    ws = jnp.concatenate([We1[D:2 * D], Wv1[D:2 * D]], axis=1)
    w1d = We1[2 * D].reshape(1, H)
    w1v = We1[2 * D + 1].reshape(1, H)
    wv1d = Wv1[2 * D].reshape(1, H)
    wv1v = Wv1[2 * D + 1].reshape(1, H)

    ts, td = _project_nodes(x, pos, vel, ws, wd)
    gs, gd = _sc_gather(ts, td, src, dst)
    s = _edge_mlp(gs, gd, w1d, w1v, be1.reshape(1, H), We2, be2.reshape(1, H),
                  We3, be3.reshape(1, H), wv1d, wv1v, bv1.reshape(1, H),
                  Wv2, bv2.reshape(1, 1))
    agg = _sc_scatter(s, dst.reshape(NW, EPW // CH, CH),
                      jnp.zeros((N, SW), jnp.float32))
    return _node_update(x, agg, Wh1[:D], Wh1[D:D + H], Wh1[D + H].reshape(1, H),
                        bh1.reshape(1, H), Wh2, bh2.reshape(1, D))


# speedup vs baseline: 9.4108x; 1.9664x over previous
import functools

import jax
import jax.numpy as jnp
from jax import lax
from jax.experimental import pallas as pl
from jax.experimental.pallas import tpu as pltpu
from jax.experimental.pallas import tpu_sc as plsc

N = 10000
E = 320000
D = 128
H = 32

NC = 2
NS = 16
NW = NC * NS

TW = 128
SW = 48

EPW = E // NW
CH = 80
SUB = 5
BLK = CH * SUB
PH = 5
IPH = EPW // PH
BPP = EPW // PH // BLK

BE = 8000


def _softplus(z):
    return jnp.maximum(z, 0.0) + jnp.log1p(jnp.exp(-jnp.abs(z)))


def _proj_body(x_ref, pos_ref, vel_ref, ws_ref, wd_ref, ts_ref, td_ref):
    xx = x_ref[...]
    zpad = jnp.zeros((xx.shape[0], TW - 68), jnp.float32)
    ts_ref[...] = jnp.concatenate(
        [jnp.dot(xx, ws_ref[...], preferred_element_type=jnp.float32),
         pos_ref[...], vel_ref[...], zpad], axis=1)
    td_ref[...] = jnp.concatenate(
        [jnp.dot(xx, wd_ref[...], preferred_element_type=jnp.float32),
         -pos_ref[...], -vel_ref[...], zpad], axis=1)


def _project_nodes(x, pos, vel, ws, wd):
    return pl.pallas_call(
        _proj_body,
        out_shape=(jax.ShapeDtypeStruct((N, TW), jnp.float32),
                   jax.ShapeDtypeStruct((N, TW), jnp.float32)),
    )(x, pos, vel, ws, wd)


def _sc_gather_body(ts_hbm, td_hbm, src_hbm, dst_hbm, gs_hbm, gd_hbm,
                    srcv, dstv, bs, bd, sem_s, sem_d):
    wid = lax.axis_index("s") * NC + lax.axis_index("c")
    base = wid * EPW

    @pl.loop(0, PH)
    def _ph(ph):
        pbase = base + ph * IPH
        pltpu.sync_copy(src_hbm.at[pl.ds(pbase, IPH)], srcv)
        pltpu.sync_copy(dst_hbm.at[pl.ds(pbase, IPH)], dstv)

        @pl.loop(0, BPP)
        def _blk(b):
            off = b * BLK
            cps = []
            for i in range(SUB):
                cps.append(pltpu.async_copy(
                    ts_hbm.at[srcv.at[pl.ds(off + i * CH, CH)]],
                    bs.at[pl.ds(i * CH, CH)], sem_s))
                cps.append(pltpu.async_copy(
                    td_hbm.at[dstv.at[pl.ds(off + i * CH, CH)]],
                    bd.at[pl.ds(i * CH, CH)], sem_d))
            for c in cps:
                c.wait()
            pltpu.sync_copy(bs, gs_hbm.at[pl.ds(pbase + off, BLK)])
            pltpu.sync_copy(bd, gd_hbm.at[pl.ds(pbase + off, BLK)])


_sc_gather = functools.partial(
    pl.kernel,
    out_type=(jax.ShapeDtypeStruct((E, TW), jnp.float32),
              jax.ShapeDtypeStruct((E, TW), jnp.float32)),
    mesh=plsc.VectorSubcoreMesh(core_axis_name="c", subcore_axis_name="s"),
    scratch_types=[
        pltpu.VMEM((IPH,), jnp.int32),
        pltpu.VMEM((IPH,), jnp.int32),
        pltpu.VMEM((BLK, TW), jnp.float32),
        pltpu.VMEM((BLK, TW), jnp.float32),
        pltpu.SemaphoreType.DMA,
        pltpu.SemaphoreType.DMA,
    ],
    compiler_params=pltpu.CompilerParams(use_tc_tiling_on_sc=False),
)(_sc_gather_body)


def _edge_body(gs_ref, gd_ref, p44_ref, wp_ref, bcat_ref, w2d_ref, b2d_ref,
               w3_ref, b3_ref, bsel_ref, p43_ref, zmap_ref, mmap_ref,
               cvec_ref, out_ref):
    sm = gs_ref[...] + gd_ref[...]
    rel = sm[:, 64:68]
    rp = jnp.dot(rel, p44_ref[...], preferred_element_type=jnp.float32)
    p = rel * rp
    a1 = (sm[:, 0:64]
          + jnp.dot(p, wp_ref[...], preferred_element_type=jnp.float32)
          + bcat_ref[...])
    hv1 = _softplus(a1)
    a2 = jnp.dot(hv1, w2d_ref[...], preferred_element_type=jnp.float32) + b2d_ref[...]
    s2 = _softplus(a2)
    lane = lax.broadcasted_iota(jnp.int32, a2.shape, 1)
    hv2 = jnp.where(lane < H, s2, a2)
    z = jnp.dot(hv2, w3_ref[...], preferred_element_type=jnp.float32) + b3_ref[...]
    vwb = jnp.dot(z, bsel_ref[...], preferred_element_type=jnp.float32)
    rxy = jnp.dot(rel, p43_ref[...], preferred_element_type=jnp.float32)
    mc = vwb * rxy
    out_ref[...] = (jnp.dot(z, zmap_ref[...], preferred_element_type=jnp.float32)
                    + jnp.dot(mc, mmap_ref[...], preferred_element_type=jnp.float32)
                    + cvec_ref[...])


def _edge_mlp(gs, gd, p44, wp, bcat, w2d, b2d, w3, b3, bsel, p43, zmap, mmap,
              cvec):
    full = lambda s: pl.BlockSpec(s, lambda i: (0,) * len(s))
    return pl.pallas_call(
        _edge_body,
        grid=(E // BE,),
        in_specs=[
            pl.BlockSpec((BE, TW), lambda i: (i, 0)),
            pl.BlockSpec((BE, TW), lambda i: (i, 0)),
            full((4, 4)), full((4, 2 * H)), full((1, 2 * H)),
            full((2 * H, 2 * H)), full((1, 2 * H)),
            full((2 * H, 2 * H)), full((1, 2 * H)),
            full((2 * H, 8)), full((4, 8)),
            full((2 * H, TW)), full((8, TW)), full((1, TW)),
        ],
        out_specs=pl.BlockSpec((BE, TW), lambda i: (i, 0)),
        out_shape=jax.ShapeDtypeStruct((E, TW), jnp.float32),
    )(gs, gd, p44, wp, bcat, w2d, b2d, w3, b3, bsel, p43, zmap, mmap, cvec)


NROWS_PS = N // NS


def _sc_scatter_body(s_hbm, dsti_hbm, zeros_hbm, agg_hbm, idxv, sv, shared, sem):
    cid = lax.axis_index("c")
    sid = lax.axis_index("s")
    wid = sid * NC + cid
    pltpu.sync_copy(zeros_hbm.at[pl.ds(sid * NROWS_PS, NROWS_PS)],
                    shared.at[pl.ds(sid * NROWS_PS, NROWS_PS)])
    plsc.subcore_barrier()
    base = wid * EPW
    pltpu.sync_copy(dsti_hbm.at[wid], idxv)

    @pl.loop(0, EPW // BLK)
    def _blk(b):
        pltpu.sync_copy(s_hbm.at[pl.ds(base + b * BLK, BLK), pl.ds(0, SW)], sv)
        for i in range(SUB):
            pltpu.sync_copy(sv.at[pl.ds(i * CH, CH)],
                            shared.at[idxv.at[b * SUB + i]], add=True)

    plsc.subcore_barrier()
    pltpu.sync_copy(shared.at[pl.ds(sid * NROWS_PS, NROWS_PS)],
                    agg_hbm.at[cid, pl.ds(sid * NROWS_PS, NROWS_PS)])


_sc_scatter = functools.partial(
    pl.kernel,
    out_type=jax.ShapeDtypeStruct((NC, N, SW), jnp.float32),
    mesh=plsc.VectorSubcoreMesh(core_axis_name="c", subcore_axis_name="s"),
    scratch_types=[
        pltpu.VMEM((EPW // CH, CH), jnp.int32),
        pltpu.VMEM((BLK, SW), jnp.float32),
        pltpu.VMEM_SHARED((N, SW), jnp.float32),
        pltpu.SemaphoreType.DMA,
    ],
    compiler_params=pltpu.CompilerParams(use_tc_tiling_on_sc=False),
)(_sc_scatter_body)


def _node_body(x_ref, agg_ref, wh1x_ref, wh1m_ref, wh1n_ref, bh1_ref,
               wh2_ref, bh2_ref, out_ref):
    agg = agg_ref[0] + agg_ref[1]
    cnt = jnp.maximum(agg[:, 34:35], 1.0)
    mh = agg[:, 0:32] / cnt
    mvx = agg[:, 32:33] / cnt
    mvy = agg[:, 33:34] / cnt
    norm = jnp.sqrt(mvx * mvx + mvy * mvy + 1e-12)
    xx = x_ref[...]
    pre = (jnp.dot(xx, wh1x_ref[...], preferred_element_type=jnp.float32)
           + jnp.dot(mh, wh1m_ref[...], preferred_element_type=jnp.float32)
           + norm * wh1n_ref[...] + bh1_ref[...])
    hu = _softplus(pre)
    out_ref[...] = (xx + jnp.dot(hu, wh2_ref[...], preferred_element_type=jnp.float32)
                    + bh2_ref[...])


def _node_update(x, agg, wh1x, wh1m, wh1n, bh1, wh2, bh2):
    return pl.pallas_call(
        _node_body,
        out_shape=jax.ShapeDtypeStruct((N, D), jnp.float32),
    )(x, agg, wh1x, wh1m, wh1n, bh1, wh2, bh2)


def kernel(x, pos, vel, edge_index, We1, be1, We2, be2, We3, be3,
           Wv1, bv1, Wv2, bv2, Wh1, bh1, Wh2, bh2):
    src = edge_index[0]
    dst = edge_index[1]
    f32 = jnp.float32

    wd = jnp.concatenate([We1[:D], Wv1[:D]], axis=1)
    ws = jnp.concatenate([We1[D:2 * D], Wv1[D:2 * D]], axis=1)
    w1d = We1[2 * D]
    w1v = We1[2 * D + 1]
    wv1d = Wv1[2 * D]
    wv1v = Wv1[2 * D + 1]

    p44 = jnp.zeros((4, 4), f32).at[0, 0].set(1.).at[1, 1].set(1.) \
        .at[0, 2].set(1.).at[1, 3].set(1.)
    wp = jnp.concatenate([
        jnp.concatenate([w1d, wv1d]).reshape(1, 2 * H),
        jnp.concatenate([w1d, wv1d]).reshape(1, 2 * H),
        jnp.concatenate([w1v, wv1v]).reshape(1, 2 * H),
        jnp.concatenate([w1v, wv1v]).reshape(1, 2 * H)], axis=0)
    bcat = jnp.concatenate([be1, bv1]).reshape(1, 2 * H)

    w2d = jnp.zeros((2 * H, 2 * H), f32).at[:H, :H].set(We2) \
        .at[H:, H:].set(jnp.eye(H, dtype=f32))
    b2d = jnp.concatenate([be2, jnp.zeros((H,), f32)]).reshape(1, 2 * H)

    w3 = jnp.zeros((2 * H, 2 * H), f32).at[:H, :H].set(We3) \
        .at[H:, H].set(Wv2[:, 0])
    b3 = jnp.zeros((1, 2 * H), f32).at[0, :H].set(be3).at[0, H].set(bv2[0])

    bsel = jnp.zeros((2 * H, 8), f32).at[H, 0].set(1.).at[H, 1].set(1.)
    p43 = jnp.zeros((4, 8), f32).at[0, 0].set(1.).at[1, 1].set(1.)
    zmap = jnp.zeros((2 * H, TW), f32).at[:H, :H].set(jnp.eye(H, dtype=f32))
    mmap = jnp.zeros((8, TW), f32).at[0, H].set(1.).at[1, H + 1].set(1.)
    cvec = jnp.zeros((1, TW), f32).at[0, H + 2].set(1.)

    ts, td = _project_nodes(x, pos, vel, ws, wd)
    gs, gd = _sc_gather(ts, td, src, dst)
    s = _edge_mlp(gs, gd, p44, wp, bcat, w2d, b2d, w3, b3, bsel, p43,
                  zmap, mmap, cvec)
    agg = _sc_scatter(s, dst.reshape(NW, EPW // CH, CH),
                      jnp.zeros((N, SW), f32))
    return _node_update(x, agg, Wh1[:D], Wh1[D:D + H], Wh1[D + H].reshape(1, H),
                        bh1.reshape(1, H), Wh2, bh2.reshape(1, D))


# speedup vs baseline: 10.2333x; 1.0874x over previous
import functools

import jax
import jax.numpy as jnp
from jax import lax
from jax.experimental import pallas as pl
from jax.experimental.pallas import tpu as pltpu
from jax.experimental.pallas import tpu_sc as plsc

N = 10000
E = 320000
D = 128
H = 32

NC = 2
NS = 16
NW = NC * NS

TW = 80
GW = 128
SW = 48

EPW = E // NW
CH = 80
SUB = 5
BLK = CH * SUB
PH = 5
IPH = EPW // PH
BPP = EPW // PH // BLK

BE = 8000


def _softplus(z):
    return jnp.maximum(z, 0.0) + jnp.log1p(jnp.exp(-jnp.abs(z)))


def _proj_body(x_ref, pos_ref, vel_ref, ws_ref, wd_ref, ts_ref, td_ref):
    xx = x_ref[...]
    zpad = jnp.zeros((xx.shape[0], TW - 68), jnp.float32)
    ts_ref[...] = jnp.concatenate(
        [jnp.dot(xx, ws_ref[...], preferred_element_type=jnp.float32),
         pos_ref[...], vel_ref[...], zpad], axis=1)
    td_ref[...] = jnp.concatenate(
        [jnp.dot(xx, wd_ref[...], preferred_element_type=jnp.float32),
         -pos_ref[...], -vel_ref[...], zpad], axis=1)


def _project_nodes(x, pos, vel, ws, wd):
    return pl.pallas_call(
        _proj_body,
        out_shape=(jax.ShapeDtypeStruct((N, TW), jnp.float32),
                   jax.ShapeDtypeStruct((N, TW), jnp.float32)),
    )(x, pos, vel, ws, wd)


def _sc_gather_body(ts_hbm, td_hbm, src_hbm, dst_hbm, gs_hbm, gd_hbm,
                    srcv, dstv, bs, bd, sem_s, sem_d):
    wid = lax.axis_index("s") * NC + lax.axis_index("c")
    base = wid * EPW

    @pl.loop(0, PH)
    def _ph(ph):
        pbase = base + ph * IPH
        pltpu.sync_copy(src_hbm.at[pl.ds(pbase, IPH)], srcv)
        pltpu.sync_copy(dst_hbm.at[pl.ds(pbase, IPH)], dstv)

        @pl.loop(0, BPP)
        def _blk(b):
            off = b * BLK
            cps = []
            for i in range(SUB):
                cps.append(pltpu.async_copy(
                    ts_hbm.at[srcv.at[pl.ds(off + i * CH, CH)]],
                    bs.at[pl.ds(i * CH, CH)], sem_s))
                cps.append(pltpu.async_copy(
                    td_hbm.at[dstv.at[pl.ds(off + i * CH, CH)]],
                    bd.at[pl.ds(i * CH, CH)], sem_d))
            for c in cps:
                c.wait()
            pltpu.sync_copy(bs, gs_hbm.at[pl.ds(pbase + off, BLK), pl.ds(0, TW)])
            pltpu.sync_copy(bd, gd_hbm.at[pl.ds(pbase + off, BLK), pl.ds(0, TW)])


_sc_gather = functools.partial(
    pl.kernel,
    out_type=(jax.ShapeDtypeStruct((E, GW), jnp.float32),
              jax.ShapeDtypeStruct((E, GW), jnp.float32)),
    mesh=plsc.VectorSubcoreMesh(core_axis_name="c", subcore_axis_name="s"),
    scratch_types=[
        pltpu.VMEM((IPH,), jnp.int32),
        pltpu.VMEM((IPH,), jnp.int32),
        pltpu.VMEM((BLK, TW), jnp.float32),
        pltpu.VMEM((BLK, TW), jnp.float32),
        pltpu.SemaphoreType.DMA,
        pltpu.SemaphoreType.DMA,
    ],
    compiler_params=pltpu.CompilerParams(use_tc_tiling_on_sc=False),
)(_sc_gather_body)


def _edge_body(gs_ref, gd_ref, p44_ref, wp_ref, bcat_ref, w2d_ref, b2d_ref,
               w3_ref, b3_ref, bsel_ref, p43_ref, zmap_ref, mmap_ref,
               cvec_ref, out_ref):
    sm = gs_ref[...] + gd_ref[...]
    rel = sm[:, 64:68]
    rp = jnp.dot(rel, p44_ref[...], preferred_element_type=jnp.float32)
    p = rel * rp
    a1 = (sm[:, 0:64]
          + jnp.dot(p, wp_ref[...], preferred_element_type=jnp.float32)
          + bcat_ref[...])
    hv1 = _softplus(a1)
    a2 = jnp.dot(hv1, w2d_ref[...], preferred_element_type=jnp.float32) + b2d_ref[...]
    s2 = _softplus(a2)
    lane = lax.broadcasted_iota(jnp.int32, a2.shape, 1)
    hv2 = jnp.where(lane < H, s2, a2)
    z = jnp.dot(hv2, w3_ref[...], preferred_element_type=jnp.float32) + b3_ref[...]
    vwb = jnp.dot(z, bsel_ref[...], preferred_element_type=jnp.float32)
    rxy = jnp.dot(rel, p43_ref[...], preferred_element_type=jnp.float32)
    mc = vwb * rxy
    out_ref[...] = (jnp.dot(z, zmap_ref[...], preferred_element_type=jnp.float32)
                    + jnp.dot(mc, mmap_ref[...], preferred_element_type=jnp.float32)
                    + cvec_ref[...])


def _edge_mlp(gs, gd, p44, wp, bcat, w2d, b2d, w3, b3, bsel, p43, zmap, mmap,
              cvec):
    full = lambda s: pl.BlockSpec(s, lambda i: (0,) * len(s))
    return pl.pallas_call(
        _edge_body,
        grid=(E // BE,),
        in_specs=[
            pl.BlockSpec((BE, GW), lambda i: (i, 0)),
            pl.BlockSpec((BE, GW), lambda i: (i, 0)),
            full((4, 4)), full((4, 2 * H)), full((1, 2 * H)),
            full((2 * H, 2 * H)), full((1, 2 * H)),
            full((2 * H, 2 * H)), full((1, 2 * H)),
            full((2 * H, 8)), full((4, 8)),
            full((2 * H, GW)), full((8, GW)), full((1, GW)),
        ],
        out_specs=pl.BlockSpec((BE, GW), lambda i: (i, 0)),
        out_shape=jax.ShapeDtypeStruct((E, GW), jnp.float32),
    )(gs, gd, p44, wp, bcat, w2d, b2d, w3, b3, bsel, p43, zmap, mmap, cvec)


NROWS_PS = N // NS


def _sc_scatter_body(s_hbm, dsti_hbm, zeros_hbm, agg_hbm, idxv, sv, shared, sem):
    cid = lax.axis_index("c")
    sid = lax.axis_index("s")
    wid = sid * NC + cid
    pltpu.sync_copy(zeros_hbm.at[pl.ds(sid * NROWS_PS, NROWS_PS)],
                    shared.at[pl.ds(sid * NROWS_PS, NROWS_PS)])
    plsc.subcore_barrier()
    base = wid * EPW
    pltpu.sync_copy(dsti_hbm.at[wid], idxv)

    @pl.loop(0, EPW // BLK)
    def _blk(b):
        pltpu.sync_copy(s_hbm.at[pl.ds(base + b * BLK, BLK), pl.ds(0, SW)], sv)
        for i in range(SUB):
            pltpu.sync_copy(sv.at[pl.ds(i * CH, CH)],
                            shared.at[idxv.at[b * SUB + i]], add=True)

    plsc.subcore_barrier()
    pltpu.sync_copy(shared.at[pl.ds(sid * NROWS_PS, NROWS_PS)],
                    agg_hbm.at[cid, pl.ds(sid * NROWS_PS, NROWS_PS)])


_sc_scatter = functools.partial(
    pl.kernel,
    out_type=jax.ShapeDtypeStruct((NC, N, SW), jnp.float32),
    mesh=plsc.VectorSubcoreMesh(core_axis_name="c", subcore_axis_name="s"),
    scratch_types=[
        pltpu.VMEM((EPW // CH, CH), jnp.int32),
        pltpu.VMEM((BLK, SW), jnp.float32),
        pltpu.VMEM_SHARED((N, SW), jnp.float32),
        pltpu.SemaphoreType.DMA,
    ],
    compiler_params=pltpu.CompilerParams(use_tc_tiling_on_sc=False),
)(_sc_scatter_body)


def _node_body(x_ref, agg_ref, wh1x_ref, wh1m_ref, wh1n_ref, bh1_ref,
               wh2_ref, bh2_ref, out_ref):
    agg = agg_ref[0] + agg_ref[1]
    cnt = jnp.maximum(agg[:, 34:35], 1.0)
    mh = agg[:, 0:32] / cnt
    mvx = agg[:, 32:33] / cnt
    mvy = agg[:, 33:34] / cnt
    norm = jnp.sqrt(mvx * mvx + mvy * mvy + 1e-12)
    xx = x_ref[...]
    pre = (jnp.dot(xx, wh1x_ref[...], preferred_element_type=jnp.float32)
           + jnp.dot(mh, wh1m_ref[...], preferred_element_type=jnp.float32)
           + norm * wh1n_ref[...] + bh1_ref[...])
    hu = _softplus(pre)
    out_ref[...] = (xx + jnp.dot(hu, wh2_ref[...], preferred_element_type=jnp.float32)
                    + bh2_ref[...])


def _node_update(x, agg, wh1x, wh1m, wh1n, bh1, wh2, bh2):
    return pl.pallas_call(
        _node_body,
        out_shape=jax.ShapeDtypeStruct((N, D), jnp.float32),
    )(x, agg, wh1x, wh1m, wh1n, bh1, wh2, bh2)


def kernel(x, pos, vel, edge_index, We1, be1, We2, be2, We3, be3,
           Wv1, bv1, Wv2, bv2, Wh1, bh1, Wh2, bh2):
    src = edge_index[0]
    dst = edge_index[1]
    f32 = jnp.float32

    wd = jnp.concatenate([We1[:D], Wv1[:D]], axis=1)
    ws = jnp.concatenate([We1[D:2 * D], Wv1[D:2 * D]], axis=1)
    w1d = We1[2 * D]
    w1v = We1[2 * D + 1]
    wv1d = Wv1[2 * D]
    wv1v = Wv1[2 * D + 1]

    p44 = jnp.zeros((4, 4), f32).at[0, 0].set(1.).at[1, 1].set(1.) \
        .at[0, 2].set(1.).at[1, 3].set(1.)
    wp = jnp.concatenate([
        jnp.concatenate([w1d, wv1d]).reshape(1, 2 * H),
        jnp.concatenate([w1d, wv1d]).reshape(1, 2 * H),
        jnp.concatenate([w1v, wv1v]).reshape(1, 2 * H),
        jnp.concatenate([w1v, wv1v]).reshape(1, 2 * H)], axis=0)
    bcat = jnp.concatenate([be1, bv1]).reshape(1, 2 * H)

    w2d = jnp.zeros((2 * H, 2 * H), f32).at[:H, :H].set(We2) \
        .at[H:, H:].set(jnp.eye(H, dtype=f32))
    b2d = jnp.concatenate([be2, jnp.zeros((H,), f32)]).reshape(1, 2 * H)

    w3 = jnp.zeros((2 * H, 2 * H), f32).at[:H, :H].set(We3) \
        .at[H:, H].set(Wv2[:, 0])
    b3 = jnp.zeros((1, 2 * H), f32).at[0, :H].set(be3).at[0, H].set(bv2[0])

    bsel = jnp.zeros((2 * H, 8), f32).at[H, 0].set(1.).at[H, 1].set(1.)
    p43 = jnp.zeros((4, 8), f32).at[0, 0].set(1.).at[1, 1].set(1.)
    zmap = jnp.zeros((2 * H, GW), f32).at[:H, :H].set(jnp.eye(H, dtype=f32))
    mmap = jnp.zeros((8, GW), f32).at[0, H].set(1.).at[1, H + 1].set(1.)
    cvec = jnp.zeros((1, GW), f32).at[0, H + 2].set(1.)

    ts, td = _project_nodes(x, pos, vel, ws, wd)
    gs, gd = _sc_gather(ts, td, src, dst)
    s = _edge_mlp(gs, gd, p44, wp, bcat, w2d, b2d, w3, b3, bsel, p43,
                  zmap, mmap, cvec)
    agg = _sc_scatter(s, dst.reshape(NW, EPW // CH, CH),
                      jnp.zeros((N, SW), f32))
    return _node_update(x, agg, Wh1[:D], Wh1[D:D + H], Wh1[D + H].reshape(1, H),
                        bh1.reshape(1, H), Wh2, bh2.reshape(1, D))


# speedup vs baseline: 10.9002x; 1.0652x over previous
import functools

import jax
import jax.numpy as jnp
from jax import lax
from jax.experimental import pallas as pl
from jax.experimental.pallas import tpu as pltpu
from jax.experimental.pallas import tpu_sc as plsc

N = 10000
E = 320000
D = 128
H = 32

NC = 2
NS = 16
NW = NC * NS

TW = 80
GW = 128
SW = 48

EPW = E // NW
CH = 80
SUB = 5
BLK = CH * SUB
PH = 5
IPH = EPW // PH
BPP = EPW // PH // BLK

BE = 8000


def _softplus(z):
    return jnp.maximum(z, 0.0) + jnp.log1p(jnp.exp(-jnp.abs(z)))


def _sp2(z):
    return jnp.maximum(z, 0.0) + jnp.log2(1.0 + jnp.exp2(-jnp.abs(z)))


def _proj_body(x_ref, pos_ref, vel_ref, ws_ref, wd_ref, ts_ref, td_ref):
    xx = x_ref[...]
    zpad = jnp.zeros((xx.shape[0], TW - 68), jnp.float32)
    ts_ref[...] = jnp.concatenate(
        [jnp.dot(xx, ws_ref[...], preferred_element_type=jnp.float32),
         pos_ref[...], vel_ref[...], zpad], axis=1)
    td_ref[...] = jnp.concatenate(
        [jnp.dot(xx, wd_ref[...], preferred_element_type=jnp.float32),
         -pos_ref[...], -vel_ref[...], zpad], axis=1)


def _project_nodes(x, pos, vel, ws, wd):
    return pl.pallas_call(
        _proj_body,
        out_shape=(jax.ShapeDtypeStruct((N, TW), jnp.float32),
                   jax.ShapeDtypeStruct((N, TW), jnp.float32)),
    )(x, pos, vel, ws, wd)


def _sc_gather_body(ts_hbm, td_hbm, src_hbm, dst_hbm, gs_hbm, gd_hbm,
                    srcv, dstv, bs, bd, sem_s, sem_d):
    wid = lax.axis_index("s") * NC + lax.axis_index("c")
    base = wid * EPW

    @pl.loop(0, PH)
    def _ph(ph):
        pbase = base + ph * IPH
        pltpu.sync_copy(src_hbm.at[pl.ds(pbase, IPH)], srcv)
        pltpu.sync_copy(dst_hbm.at[pl.ds(pbase, IPH)], dstv)

        @pl.loop(0, BPP)
        def _blk(b):
            off = b * BLK
            cps = []
            for i in range(SUB):
                cps.append(pltpu.async_copy(
                    ts_hbm.at[srcv.at[pl.ds(off + i * CH, CH)]],
                    bs.at[pl.ds(i * CH, CH)], sem_s))
                cps.append(pltpu.async_copy(
                    td_hbm.at[dstv.at[pl.ds(off + i * CH, CH)]],
                    bd.at[pl.ds(i * CH, CH)], sem_d))
            for c in cps:
                c.wait()
            pltpu.sync_copy(bs, gs_hbm.at[pl.ds(pbase + off, BLK), pl.ds(0, TW)])
            pltpu.sync_copy(bd, gd_hbm.at[pl.ds(pbase + off, BLK), pl.ds(0, TW)])


_sc_gather = functools.partial(
    pl.kernel,
    out_type=(jax.ShapeDtypeStruct((E, GW), jnp.float32),
              jax.ShapeDtypeStruct((E, GW), jnp.float32)),
    mesh=plsc.VectorSubcoreMesh(core_axis_name="c", subcore_axis_name="s"),
    scratch_types=[
        pltpu.VMEM((IPH,), jnp.int32),
        pltpu.VMEM((IPH,), jnp.int32),
        pltpu.VMEM((BLK, TW), jnp.float32),
        pltpu.VMEM((BLK, TW), jnp.float32),
        pltpu.SemaphoreType.DMA,
        pltpu.SemaphoreType.DMA,
    ],
    compiler_params=pltpu.CompilerParams(use_tc_tiling_on_sc=False),
)(_sc_gather_body)


def _edge_body(gs_ref, gd_ref, p44_ref, wp_ref, bcat_ref, w2d_ref, b2d_ref,
               w3z_ref, w3b_ref, bvc_ref, p43_ref, mmap_ref, cvec_ref,
               out_ref):
    sm = gs_ref[...] + gd_ref[...]
    rel = sm[:, 64:68]
    rp = jnp.dot(rel, p44_ref[...], preferred_element_type=jnp.float32)
    p = rel * rp
    a1 = (sm[:, 0:64]
          + jnp.dot(p, wp_ref[...], preferred_element_type=jnp.float32)
          + bcat_ref[...])
    hv1 = _sp2(a1)
    a2 = jnp.dot(hv1, w2d_ref[...], preferred_element_type=jnp.float32) + b2d_ref[...]
    s2 = _sp2(a2)
    lane = lax.broadcasted_iota(jnp.int32, a2.shape, 1)
    hv2 = jnp.where(lane < H, s2, a2)
    vwb = jnp.dot(hv2, w3b_ref[...], preferred_element_type=jnp.float32) + bvc_ref[...]
    rxy = jnp.dot(rel, p43_ref[...], preferred_element_type=jnp.float32)
    mc = vwb * rxy
    out_ref[...] = (jnp.dot(hv2, w3z_ref[...], preferred_element_type=jnp.float32)
                    + jnp.dot(mc, mmap_ref[...], preferred_element_type=jnp.float32)
                    + cvec_ref[...])


def _edge_mlp(gs, gd, p44, wp, bcat, w2d, b2d, w3z, w3b, bvc, p43, mmap,
              cvec):
    full = lambda s: pl.BlockSpec(s, lambda i: (0,) * len(s))
    return pl.pallas_call(
        _edge_body,
        grid=(E // BE,),
        in_specs=[
            pl.BlockSpec((BE, GW), lambda i: (i, 0)),
            pl.BlockSpec((BE, GW), lambda i: (i, 0)),
            full((4, 4)), full((4, 2 * H)), full((1, 2 * H)),
            full((2 * H, 2 * H)), full((1, 2 * H)),
            full((2 * H, GW)), full((2 * H, 8)), full((1, 8)),
            full((4, 8)), full((8, GW)), full((1, GW)),
        ],
        out_specs=pl.BlockSpec((BE, GW), lambda i: (i, 0)),
        out_shape=jax.ShapeDtypeStruct((E, GW), jnp.float32),
    )(gs, gd, p44, wp, bcat, w2d, b2d, w3z, w3b, bvc, p43, mmap, cvec)


NROWS_PS = N // NS


def _sc_scatter_body(s_hbm, dsti_hbm, zeros_hbm, agg_hbm, idxv, sv, shared, sem):
    cid = lax.axis_index("c")
    sid = lax.axis_index("s")
    wid = sid * NC + cid
    pltpu.sync_copy(zeros_hbm.at[pl.ds(sid * NROWS_PS, NROWS_PS)],
                    shared.at[pl.ds(sid * NROWS_PS, NROWS_PS)])
    plsc.subcore_barrier()
    base = wid * EPW
    pltpu.sync_copy(dsti_hbm.at[wid], idxv)

    @pl.loop(0, EPW // BLK)
    def _blk(b):
        pltpu.sync_copy(s_hbm.at[pl.ds(base + b * BLK, BLK), pl.ds(0, SW)], sv)
        for i in range(SUB):
            pltpu.sync_copy(sv.at[pl.ds(i * CH, CH)],
                            shared.at[idxv.at[b * SUB + i]], add=True)

    plsc.subcore_barrier()
    pltpu.sync_copy(shared.at[pl.ds(sid * NROWS_PS, NROWS_PS)],
                    agg_hbm.at[cid, pl.ds(sid * NROWS_PS, NROWS_PS)])


_sc_scatter = functools.partial(
    pl.kernel,
    out_type=jax.ShapeDtypeStruct((NC, N, SW), jnp.float32),
    mesh=plsc.VectorSubcoreMesh(core_axis_name="c", subcore_axis_name="s"),
    scratch_types=[
        pltpu.VMEM((EPW // CH, CH), jnp.int32),
        pltpu.VMEM((BLK, SW), jnp.float32),
        pltpu.VMEM_SHARED((N, SW), jnp.float32),
        pltpu.SemaphoreType.DMA,
    ],
    compiler_params=pltpu.CompilerParams(use_tc_tiling_on_sc=False),
)(_sc_scatter_body)


def _node_body(x_ref, agg_ref, wh1x_ref, wh1m_ref, wh1n_ref, bh1_ref,
               wh2_ref, bh2_ref, out_ref):
    agg = agg_ref[0] + agg_ref[1]
    cnt = jnp.maximum(agg[:, 34:35], 1.0)
    mh = agg[:, 0:32] / cnt
    mvx = agg[:, 32:33] / cnt
    mvy = agg[:, 33:34] / cnt
    norm = jnp.sqrt(mvx * mvx + mvy * mvy + 1e-12)
    xx = x_ref[...]
    pre = (jnp.dot(xx, wh1x_ref[...], preferred_element_type=jnp.float32)
           + jnp.dot(mh, wh1m_ref[...], preferred_element_type=jnp.float32)
           + norm * wh1n_ref[...] + bh1_ref[...])
    hu = _softplus(pre)
    out_ref[...] = (xx + jnp.dot(hu, wh2_ref[...], preferred_element_type=jnp.float32)
                    + bh2_ref[...])


def _node_update(x, agg, wh1x, wh1m, wh1n, bh1, wh2, bh2):
    return pl.pallas_call(
        _node_body,
        out_shape=jax.ShapeDtypeStruct((N, D), jnp.float32),
    )(x, agg, wh1x, wh1m, wh1n, bh1, wh2, bh2)


def kernel(x, pos, vel, edge_index, We1, be1, We2, be2, We3, be3,
           Wv1, bv1, Wv2, bv2, Wh1, bh1, Wh2, bh2):
    src = edge_index[0]
    dst = edge_index[1]
    f32 = jnp.float32

    wd = jnp.concatenate([We1[:D], Wv1[:D]], axis=1)
    ws = jnp.concatenate([We1[D:2 * D], Wv1[D:2 * D]], axis=1)
    w1d = We1[2 * D]
    w1v = We1[2 * D + 1]
    wv1d = Wv1[2 * D]
    wv1v = Wv1[2 * D + 1]

    lg2e = f32(1.4426950408889634)
    ln2 = f32(0.6931471805599453)

    p44 = jnp.zeros((4, 4), f32).at[0, 0].set(1.).at[1, 1].set(1.) \
        .at[0, 2].set(1.).at[1, 3].set(1.)
    wp = lg2e * jnp.concatenate([
        jnp.concatenate([w1d, wv1d]).reshape(1, 2 * H),
        jnp.concatenate([w1d, wv1d]).reshape(1, 2 * H),
        jnp.concatenate([w1v, wv1v]).reshape(1, 2 * H),
        jnp.concatenate([w1v, wv1v]).reshape(1, 2 * H)], axis=0)
    bcat = lg2e * jnp.concatenate([be1, bv1]).reshape(1, 2 * H)

    w2d = jnp.zeros((2 * H, 2 * H), f32).at[:H, :H].set(We2) \
        .at[H:, H:].set(ln2 * jnp.eye(H, dtype=f32))
    b2d = jnp.concatenate([lg2e * be2, jnp.zeros((H,), f32)]).reshape(1, 2 * H)

    w3z = jnp.zeros((2 * H, GW), f32).at[:H, :H].set(ln2 * We3)
    w3b = jnp.zeros((2 * H, 8), f32).at[H:, 0].set(Wv2[:, 0]) \
        .at[H:, 1].set(Wv2[:, 0])
    bvc = jnp.zeros((1, 8), f32).at[0, 0].set(bv2[0]).at[0, 1].set(bv2[0])
    p43 = jnp.zeros((4, 8), f32).at[0, 0].set(1.).at[1, 1].set(1.)
    mmap = jnp.zeros((8, GW), f32).at[0, H].set(1.).at[1, H + 1].set(1.)
    cvec = jnp.zeros((1, GW), f32).at[0, H + 2].set(1.) \
        .at[0, :H].set(be3)

    ts, td = _project_nodes(x, pos, vel, lg2e * ws, lg2e * wd)
    gs, gd = _sc_gather(ts, td, src, dst)
    s = _edge_mlp(gs, gd, p44, wp, bcat, w2d, b2d, w3z, w3b, bvc, p43,
                  mmap, cvec)
    agg = _sc_scatter(s, dst.reshape(NW, EPW // CH, CH),
                      jnp.zeros((N, SW), f32))
    return _node_update(x, agg, Wh1[:D], Wh1[D:D + H], Wh1[D + H].reshape(1, H),
                        bh1.reshape(1, H), Wh2, bh2.reshape(1, D))
